# baseline (speedup 1.0000x reference)
import functools
import numpy as np
import jax
import jax.numpy as jnp
from jax import lax
from jax.experimental import pallas as pl
from jax.experimental.pallas import tpu as pltpu
from jax.experimental.pallas import tpu_sc as plsc

_C = 26
_D = 32
_B = 16384
_CARD = 100000
_NW = 32
_TOTAL = _B * _C
_PER_W = _TOTAL // _NW
_CHUNK = 832
_NCHUNK = _PER_W // _CHUNK
_L = 16

_OFF_EXP = np.asarray(
    [(p % _C) * _CARD for p in range(208)], dtype=np.int32)


def _body(x_ref, tab_ref, bias_ref, off_ref, out_ref,
          idx_v, off_v, bias_v, rows0, rows1,
          gsem0, gsem1, osem0, osem1):
  cid = lax.axis_index("c")
  sid = lax.axis_index("s")
  wid = sid * 2 + cid
  base = wid * _PER_W

  pltpu.sync_copy(x_ref.at[pl.ds(base, _PER_W)], idx_v)
  pltpu.sync_copy(bias_ref, bias_v)
  pltpu.sync_copy(off_ref, off_v)

  def offs_body(k, carry):
    s = k * _L
    o = off_v[pl.ds(lax.rem(k, 13) * _L, _L)]
    idx_v[pl.ds(s, _L)] = idx_v[pl.ds(s, _L)] + o
    return carry
  lax.fori_loop(0, _PER_W // _L, offs_body, 0)

  rows_bufs = (rows0, rows1)
  gsems = (gsem0, gsem1)
  osems = (osem0, osem1)
  ghandles = [None, None]
  ohandles = [None, None]

  def start_gather(g):
    cur = g % 2
    ghandles[cur] = pltpu.async_copy(
        tab_ref.at[idx_v.at[pl.ds(g * _CHUNK, _CHUNK)]],
        rows_bufs[cur], gsems[cur])

  def bias_add(rv):
    def c_body(c, carry):
      b0 = bias_v[c, pl.ds(0, _L)]
      b1 = bias_v[c, pl.ds(_L, _L)]
      def m_body(m, carry2):
        r = m * _C + c
        rv[r, pl.ds(0, _L)] = rv[r, pl.ds(0, _L)] + b0
        rv[r, pl.ds(_L, _L)] = rv[r, pl.ds(_L, _L)] + b1
        return carry2
      return lax.fori_loop(0, _CHUNK // _C, m_body, carry)
    lax.fori_loop(0, _C, c_body, 0)

  start_gather(0)
  for g in range(_NCHUNK):
    cur = g % 2
    nxt = 1 - cur
    if g + 1 < _NCHUNK:
      if ohandles[nxt] is not None:
        ohandles[nxt].wait()
        ohandles[nxt] = None
      start_gather(g + 1)
    ghandles[cur].wait()
    bias_add(rows_bufs[cur])
    ohandles[cur] = pltpu.async_copy(
        rows_bufs[cur],
        out_ref.at[pl.ds(base + g * _CHUNK, _CHUNK)],
        osems[cur])
  for h in ohandles:
    if h is not None:
      h.wait()


@functools.partial(
    pl.kernel,
    out_type=jax.ShapeDtypeStruct((_TOTAL, _D), jnp.float32),
    mesh=plsc.VectorSubcoreMesh(core_axis_name="c", subcore_axis_name="s"),
    compiler_params=pltpu.CompilerParams(use_tc_tiling_on_sc=False),
    scratch_types=[
        pltpu.VMEM((_PER_W,), jnp.int32),
        pltpu.VMEM((len(_OFF_EXP),), jnp.int32),
        pltpu.VMEM((_C, _D), jnp.float32),
        pltpu.VMEM((_CHUNK, _D), jnp.float32),
        pltpu.VMEM((_CHUNK, _D), jnp.float32),
        pltpu.SemaphoreType.DMA,
        pltpu.SemaphoreType.DMA,
        pltpu.SemaphoreType.DMA,
        pltpu.SemaphoreType.DMA,
    ],
)
def _sc_lookup(x_ref, tab_ref, bias_ref, off_ref, out_ref, *scratch):
  _body(x_ref, tab_ref, bias_ref, off_ref, out_ref, *scratch)


def kernel(x_cat, table, bias):
  x_flat = x_cat.astype(jnp.int32).reshape(-1)
  out = _sc_lookup(x_flat, table, bias, jnp.asarray(_OFF_EXP))
  return out.reshape(_B, _C, _D)
